# baseline (speedup 1.0000x reference)
import functools

import jax
import jax.numpy as jnp
from jax import lax
from jax.experimental import pallas as pl
from jax.experimental.pallas import tpu as pltpu
from jax.experimental.pallas import tpu_sc as plsc

N_NODES = 10000
N_EDGES = 320000
FEAT = 128

NCORES = 2
NSUB = 16
NW = NCORES * NSUB
CHUNK = 128
CPT = 80
EPT = CPT * CHUNK
E_PAD = NW * EPT
N_PAD = 10112
ROWS_PT = N_PAD // NSUB

_MESH = plsc.VectorSubcoreMesh(core_axis_name="c", subcore_axis_name="s")


_FIRE = 8


def _deg_body(dst2d_hbm, zeros_hbm, ones_hbm, out_hbm, shared, ones_v, didx_v, sem):
    cid = lax.axis_index("c")
    sid = lax.axis_index("s")
    wid = cid * NSUB + sid
    r0 = sid * ROWS_PT
    pltpu.sync_copy(zeros_hbm.at[pl.ds(r0, ROWS_PT)], shared.at[pl.ds(r0, ROWS_PT)])
    pltpu.sync_copy(ones_hbm, ones_v)
    pltpu.sync_copy(dst2d_hbm.at[pl.ds(wid * CPT, CPT)], didx_v)
    plsc.subcore_barrier()

    def group(g, carry):
        c0 = g * _FIRE
        for k in range(_FIRE):
            pltpu.async_copy(ones_v, shared.at[didx_v.at[c0 + k]], sem, add=True)
        for _ in range(_FIRE):
            pltpu.make_async_copy(ones_v, shared.at[didx_v.at[c0]], sem).wait()
        return carry

    lax.fori_loop(0, CPT // _FIRE, group, 0)
    plsc.subcore_barrier()
    pltpu.sync_copy(shared.at[pl.ds(r0, ROWS_PT)],
                    out_hbm.at[cid, pl.ds(r0, ROWS_PT)])


_deg_call = pl.kernel(
    _deg_body,
    out_type=jax.ShapeDtypeStruct((NCORES, N_PAD, FEAT), jnp.float32),
    mesh=_MESH,
    scratch_types=[
        pltpu.VMEM_SHARED((N_PAD, FEAT), jnp.float32),
        pltpu.VMEM((CHUNK, FEAT), jnp.float32),
        pltpu.VMEM((CPT, CHUNK), jnp.int32),
        pltpu.SemaphoreType.DMA,
    ],
)


def _pass_body(xs_hbm, src_hbm, dst_hbm, zeros_hbm, out_hbm,
               shared, sidx_v, didx_v, rows_v, sem):
    cid = lax.axis_index("c")
    sid = lax.axis_index("s")
    wid = cid * NSUB + sid
    r0 = sid * ROWS_PT
    base = wid * EPT
    pltpu.sync_copy(zeros_hbm.at[pl.ds(r0, ROWS_PT)], shared.at[pl.ds(r0, ROWS_PT)])
    plsc.subcore_barrier()

    def step(c, carry):
        off = base + c * CHUNK
        pltpu.sync_copy(src_hbm.at[pl.ds(off, CHUNK)], sidx_v)
        pltpu.sync_copy(dst_hbm.at[pl.ds(off, CHUNK)], didx_v)
        pltpu.async_copy(xs_hbm.at[sidx_v], rows_v, sem).wait()
        pltpu.sync_copy(rows_v, shared.at[didx_v], add=True)
        return carry

    lax.fori_loop(0, CPT, step, 0)
    plsc.subcore_barrier()
    pltpu.sync_copy(shared.at[pl.ds(r0, ROWS_PT)],
                    out_hbm.at[cid, pl.ds(r0, ROWS_PT)])


_pass_call = pl.kernel(
    _pass_body,
    out_type=jax.ShapeDtypeStruct((NCORES, N_PAD, FEAT), jnp.float32),
    mesh=_MESH,
    scratch_types=[
        pltpu.VMEM_SHARED((N_PAD, FEAT), jnp.float32),
        pltpu.VMEM((CHUNK,), jnp.int32),
        pltpu.VMEM((CHUNK,), jnp.int32),
        pltpu.VMEM((CHUNK, FEAT), jnp.float32),
        pltpu.SemaphoreType.DMA,
    ],
)

_BLK = 2000


def _dinv_of(degp_ref):
    deg = degp_ref[0, :, 0:1] + degp_ref[1, :, 0:1]
    return lax.rsqrt(deg + 1.0)


def _prep_body(degp_ref, x_ref, xs_ref):
    xs_ref[...] = x_ref[...] * _dinv_of(degp_ref)


def _mid_body(p_ref, xs_ref, degp_ref, w1_ref, b1_ref, w2_ref, ms_ref):
    dinv = _dinv_of(degp_ref)
    s = (p_ref[0] + p_ref[1] + xs_ref[...]) * dinv
    h = jnp.maximum(
        jnp.dot(s, w1_ref[...], preferred_element_type=jnp.float32) + b1_ref[...],
        0.0)
    ms_ref[...] = jnp.dot(h, w2_ref[...], preferred_element_type=jnp.float32) * dinv


def _out_body(q_ref, ms_ref, degp_ref, b2_ref, wc_ref, bc_ref, o_ref):
    dinv = _dinv_of(degp_ref)
    h2 = jnp.maximum(
        (q_ref[0] + q_ref[1] + ms_ref[...]) * dinv + b2_ref[...], 0.0)
    o_ref[...] = (
        jnp.dot(h2, wc_ref[...], preferred_element_type=jnp.float32) + bc_ref[...])


_degp_spec = pl.BlockSpec((NCORES, _BLK, FEAT), lambda i: (0, i, 0))
_part_spec = pl.BlockSpec((NCORES, _BLK, FEAT), lambda i: (0, i, 0))
_row_spec = pl.BlockSpec((_BLK, FEAT), lambda i: (i, 0))


def _full(shape):
    return pl.BlockSpec(shape, lambda i: (0,) * len(shape))


_prep_call = pl.pallas_call(
    _prep_body,
    grid=(N_NODES // _BLK,),
    in_specs=[_degp_spec, _row_spec],
    out_specs=_row_spec,
    out_shape=jax.ShapeDtypeStruct((N_NODES, FEAT), jnp.float32),
)

_mid_call = pl.pallas_call(
    _mid_body,
    grid=(N_NODES // _BLK,),
    in_specs=[_part_spec, _row_spec, _degp_spec,
              _full((FEAT, 256)), _full((1, 256)), _full((256, FEAT))],
    out_specs=_row_spec,
    out_shape=jax.ShapeDtypeStruct((N_NODES, FEAT), jnp.float32),
)

_out_call = pl.pallas_call(
    _out_body,
    grid=(N_NODES // _BLK,),
    in_specs=[_part_spec, _row_spec, _degp_spec,
              _full((1, FEAT)), _full((FEAT, FEAT)), _full((1, FEAT))],
    out_specs=_row_spec,
    out_shape=jax.ShapeDtypeStruct((N_NODES, FEAT), jnp.float32),
)


def kernel(X, edge_index, W1, b1, W2, b2, Wc, bc):
    pad = E_PAD - N_EDGES
    src_p = jnp.concatenate([edge_index[0], jnp.zeros((pad,), jnp.int32)])
    dst_p = jnp.concatenate([edge_index[1], jnp.full((pad,), N_NODES, jnp.int32)])
    dst_2d = dst_p.reshape(NW * CPT, CHUNK)
    zeros128 = jnp.zeros((N_PAD, FEAT), jnp.float32)
    ones128 = jnp.ones((CHUNK, FEAT), jnp.float32)

    degp = _deg_call(dst_2d, zeros128, ones128)
    xs = _prep_call(degp, X)
    p = _pass_call(xs, src_p, dst_p, zeros128)
    ms = _mid_call(p, xs, degp, W1, b1.reshape(1, -1), W2)
    q = _pass_call(ms, src_p, dst_p, zeros128)

    n_classes = Wc.shape[1]
    wc_pad = jnp.zeros((FEAT, FEAT), jnp.float32).at[:, :n_classes].set(Wc)
    bc_pad = jnp.zeros((1, FEAT), jnp.float32).at[:, :n_classes].set(bc)
    o = _out_call(q, ms, degp, b2.reshape(1, -1), wc_pad, bc_pad)
    return o[:, :n_classes]


# speedup vs baseline: 2.0571x; 2.0571x over previous
import functools

import jax
import jax.numpy as jnp
from jax import lax
from jax.experimental import pallas as pl
from jax.experimental.pallas import tpu as pltpu
from jax.experimental.pallas import tpu_sc as plsc

N_NODES = 10000
N_EDGES = 320000
FEAT = 128

NCORES = 2
NSUB = 16
NW = NCORES * NSUB
CHUNK = 128
CPT = 80
EPT = CPT * CHUNK
E_PAD = NW * EPT
N_PAD = 10112
ROWS_PT = N_PAD // NSUB

_MESH = plsc.VectorSubcoreMesh(core_axis_name="c", subcore_axis_name="s")


_FIRE = 8


def _deg_body(dst2d_hbm, zeros_hbm, ones_hbm, out_hbm, shared, ones_v, didx_v, sem):
    cid = lax.axis_index("c")
    sid = lax.axis_index("s")
    wid = cid * NSUB + sid
    r0 = sid * ROWS_PT
    pltpu.sync_copy(zeros_hbm.at[pl.ds(r0, ROWS_PT)], shared.at[pl.ds(r0, ROWS_PT)])
    pltpu.sync_copy(ones_hbm, ones_v)
    pltpu.sync_copy(dst2d_hbm.at[pl.ds(wid * CPT, CPT)], didx_v)
    plsc.subcore_barrier()

    def group(g, carry):
        c0 = g * _FIRE
        for k in range(_FIRE):
            pltpu.async_copy(ones_v, shared.at[didx_v.at[c0 + k]], sem, add=True)
        for _ in range(_FIRE):
            pltpu.make_async_copy(ones_v, shared.at[didx_v.at[c0]], sem).wait()
        return carry

    lax.fori_loop(0, CPT // _FIRE, group, 0)
    plsc.subcore_barrier()
    pltpu.sync_copy(shared.at[pl.ds(r0, ROWS_PT)],
                    out_hbm.at[cid, pl.ds(r0, ROWS_PT)])


_deg_call = pl.kernel(
    _deg_body,
    out_type=jax.ShapeDtypeStruct((NCORES, N_PAD, FEAT), jnp.float32),
    mesh=_MESH,
    scratch_types=[
        pltpu.VMEM_SHARED((N_PAD, FEAT), jnp.float32),
        pltpu.VMEM((CHUNK, FEAT), jnp.float32),
        pltpu.VMEM((CPT, CHUNK), jnp.int32),
        pltpu.SemaphoreType.DMA,
    ],
)


EPT_REAL = N_EDGES // NW
CPT_FULL = EPT_REAL // CHUNK
TAIL = EPT_REAL - CPT_FULL * CHUNK


def _pass_body(xs_hbm, src_hbm, dst_hbm, zeros_hbm, out_hbm,
               shared, sidx_v, didx_v, rows_v, sidx_t, didx_t, rows_t, sem):
    cid = lax.axis_index("c")
    sid = lax.axis_index("s")
    wid = cid * NSUB + sid
    r0 = sid * ROWS_PT
    base = wid * EPT_REAL
    pltpu.sync_copy(zeros_hbm.at[pl.ds(r0, ROWS_PT)], shared.at[pl.ds(r0, ROWS_PT)])
    plsc.subcore_barrier()

    def step(c, carry):
        off = base + c * CHUNK
        pltpu.sync_copy(src_hbm.at[pl.ds(off, CHUNK)], sidx_v)
        pltpu.sync_copy(dst_hbm.at[pl.ds(off, CHUNK)], didx_v)
        pltpu.async_copy(xs_hbm.at[sidx_v], rows_v, sem).wait()
        pltpu.sync_copy(rows_v, shared.at[didx_v], add=True)
        return carry

    lax.fori_loop(0, CPT_FULL, step, 0)

    off = base + CPT_FULL * CHUNK
    pltpu.sync_copy(src_hbm.at[pl.ds(off, TAIL)], sidx_t)
    pltpu.sync_copy(dst_hbm.at[pl.ds(off, TAIL)], didx_t)
    pltpu.async_copy(xs_hbm.at[sidx_t], rows_t, sem).wait()
    pltpu.sync_copy(rows_t, shared.at[didx_t], add=True)

    plsc.subcore_barrier()
    pltpu.sync_copy(shared.at[pl.ds(r0, ROWS_PT)],
                    out_hbm.at[cid, pl.ds(r0, ROWS_PT)])


_pass_call = pl.kernel(
    _pass_body,
    out_type=jax.ShapeDtypeStruct((NCORES, N_PAD, FEAT), jnp.float32),
    mesh=_MESH,
    scratch_types=[
        pltpu.VMEM_SHARED((N_PAD, FEAT), jnp.float32),
        pltpu.VMEM((CHUNK,), jnp.int32),
        pltpu.VMEM((CHUNK,), jnp.int32),
        pltpu.VMEM((CHUNK, FEAT), jnp.float32),
        pltpu.VMEM((TAIL,), jnp.int32),
        pltpu.VMEM((TAIL,), jnp.int32),
        pltpu.VMEM((TAIL, FEAT), jnp.float32),
        pltpu.SemaphoreType.DMA,
    ],
)

_BLK = 2000


def _dinv_of(degp_ref):
    deg = degp_ref[0, :, 0:1] + degp_ref[1, :, 0:1]
    return lax.rsqrt(deg + 1.0)


def _prep_body(degp_ref, x_ref, xs_ref):
    xs_ref[...] = x_ref[...] * _dinv_of(degp_ref)


def _mid_body(p_ref, xs_ref, degp_ref, w1_ref, b1_ref, w2_ref, ms_ref):
    dinv = _dinv_of(degp_ref)
    s = (p_ref[0] + p_ref[1] + xs_ref[...]) * dinv
    h = jnp.maximum(
        jnp.dot(s, w1_ref[...], preferred_element_type=jnp.float32) + b1_ref[...],
        0.0)
    ms_ref[...] = jnp.dot(h, w2_ref[...], preferred_element_type=jnp.float32) * dinv


def _out_body(q_ref, ms_ref, degp_ref, b2_ref, wc_ref, bc_ref, o_ref):
    dinv = _dinv_of(degp_ref)
    h2 = jnp.maximum(
        (q_ref[0] + q_ref[1] + ms_ref[...]) * dinv + b2_ref[...], 0.0)
    o_ref[...] = (
        jnp.dot(h2, wc_ref[...], preferred_element_type=jnp.float32) + bc_ref[...])


_degp_spec = pl.BlockSpec((NCORES, _BLK, FEAT), lambda i: (0, i, 0))
_part_spec = pl.BlockSpec((NCORES, _BLK, FEAT), lambda i: (0, i, 0))
_row_spec = pl.BlockSpec((_BLK, FEAT), lambda i: (i, 0))


def _full(shape):
    return pl.BlockSpec(shape, lambda i: (0,) * len(shape))


_prep_call = pl.pallas_call(
    _prep_body,
    grid=(N_NODES // _BLK,),
    in_specs=[_degp_spec, _row_spec],
    out_specs=_row_spec,
    out_shape=jax.ShapeDtypeStruct((N_NODES, FEAT), jnp.float32),
)

_mid_call = pl.pallas_call(
    _mid_body,
    grid=(N_NODES // _BLK,),
    in_specs=[_part_spec, _row_spec, _degp_spec,
              _full((FEAT, 256)), _full((1, 256)), _full((256, FEAT))],
    out_specs=_row_spec,
    out_shape=jax.ShapeDtypeStruct((N_NODES, FEAT), jnp.float32),
)

_out_call = pl.pallas_call(
    _out_body,
    grid=(N_NODES // _BLK,),
    in_specs=[_part_spec, _row_spec, _degp_spec,
              _full((1, FEAT)), _full((FEAT, FEAT)), _full((1, FEAT))],
    out_specs=_row_spec,
    out_shape=jax.ShapeDtypeStruct((N_NODES, FEAT), jnp.float32),
)


def kernel(X, edge_index, W1, b1, W2, b2, Wc, bc):
    pad = E_PAD - N_EDGES
    src_p = edge_index[0]
    dst_p = edge_index[1]
    dst_2d = jnp.concatenate(
        [dst_p, jnp.full((pad,), N_NODES, jnp.int32)]).reshape(NW * CPT, CHUNK)
    zeros128 = jnp.zeros((N_PAD, FEAT), jnp.float32)
    ones128 = jnp.ones((CHUNK, FEAT), jnp.float32)

    degp = _deg_call(dst_2d, zeros128, ones128)
    xs = _prep_call(degp, X)
    p = _pass_call(xs, src_p, dst_p, zeros128)
    ms = _mid_call(p, xs, degp, W1, b1.reshape(1, -1), W2)
    q = _pass_call(ms, src_p, dst_p, zeros128)

    n_classes = Wc.shape[1]
    wc_pad = jnp.zeros((FEAT, FEAT), jnp.float32).at[:, :n_classes].set(Wc)
    bc_pad = jnp.zeros((1, FEAT), jnp.float32).at[:, :n_classes].set(bc)
    o = _out_call(q, ms, degp, b2.reshape(1, -1), wc_pad, bc_pad)
    return o[:, :n_classes]
